# baseline (speedup 1.0000x reference)
import functools

import jax
import jax.numpy as jnp
from jax import lax
from jax.experimental import pallas as pl
from jax.experimental.pallas import tpu as pltpu
from jax.experimental.pallas import tpu_sc as plsc

F32 = jnp.float32
I32 = jnp.int32

NC = 2
NS = 16
L = 16
CHUNK = 128
BLK = 6


def _sc_mesh():
  return plsc.VectorSubcoreMesh(core_axis_name="c", subcore_axis_name="s")


def _make_deg_kernel(e_pad, n_pad):
  per_tile = e_pad // NS
  n_chunks = per_tile // CHUNK
  n_blocks = n_chunks // BLK
  rows_per_tile = n_pad // NS

  @functools.partial(
      pl.kernel,
      out_type=jax.ShapeDtypeStruct((NC * n_pad,), F32),
      mesh=_sc_mesh(),
      scratch_types=[
          pltpu.VMEM((CHUNK,), F32),
          pltpu.VMEM((2 * BLK, 2, CHUNK), I32),
          pltpu.VMEM((rows_per_tile,), F32),
          pltpu.VMEM_SHARED((n_pad,), F32),
          pltpu.SemaphoreType.DMA,
      ],
      compiler_params=pltpu.CompilerParams(use_tc_tiling_on_sc=False),
  )
  def deg_kernel(degidx, deg_out, onesbuf, ijb, tbuf, acc, ssem):
    c = lax.axis_index("c")
    s = lax.axis_index("s")
    zeros16 = jnp.zeros((L,), F32)
    ones16 = jnp.ones((L,), F32)
    for j in range(CHUNK // L):
      onesbuf[pl.ds(j * L, L)] = ones16

    def zt(r, carry):
      tbuf[pl.ds(r * L, L)] = zeros16
      return carry
    lax.fori_loop(0, rows_per_tile // L, zt, 0)

    r0 = s * rows_per_tile
    pltpu.sync_copy(tbuf, acc.at[pl.ds(r0, rows_per_tile)])
    plsc.subcore_barrier()

    cbase = s * n_chunks

    def scat(rowk):
      pltpu.async_copy(onesbuf, acc.at[ijb.at[rowk, c]], ssem, add=True)

    def drain():
      pltpu.make_async_copy(onesbuf, acc.at[ijb.at[0, 0]], ssem).wait()

    pltpu.sync_copy(degidx.at[pl.ds(cbase, BLK)], ijb.at[pl.ds(0, BLK)])

    def body(j, carry):
      h = (j % 2) * BLK
      h2 = ((j + 1) % 2) * BLK
      for p in range(BLK):
        if p < 2:
          @pl.when(j > 0)
          def _():
            drain()
        else:
          drain()
        scat(h + p)
        if p == 1:
          @pl.when(j < n_blocks - 1)
          def _():
            pltpu.sync_copy(degidx.at[pl.ds(cbase + (j + 1) * BLK, BLK)],
                            ijb.at[pl.ds(h2, BLK)])
      return carry
    lax.fori_loop(0, n_blocks, body, 0)
    drain()
    drain()
    plsc.subcore_barrier()

    pltpu.sync_copy(acc.at[pl.ds(r0, rows_per_tile)], tbuf)
    pltpu.sync_copy(tbuf, deg_out.at[pl.ds(c * n_pad + r0, rows_per_tile)])

  return deg_kernel


def _make_msg_kernel(e_pad, n_pad):
  per_tile = e_pad // NS
  n_chunks = per_tile // CHUNK
  n_blocks = n_chunks // BLK
  rows_per_tile = n_pad // NS

  @functools.partial(
      pl.kernel,
      out_type=(jax.ShapeDtypeStruct((NC, n_pad, 64), F32),
                jax.ShapeDtypeStruct((NC, n_pad, 64), F32)),
      mesh=_sc_mesh(),
      scratch_types=[
          pltpu.VMEM((3, CHUNK, 64), F32),
          pltpu.VMEM((2, CHUNK, 64), F32),
          pltpu.VMEM((2 * BLK, 3, CHUNK), I32),
          pltpu.VMEM((32, 64), F32),
          pltpu.VMEM_SHARED((n_pad, 64), F32),
          pltpu.VMEM_SHARED((n_pad, 64), F32),
          pltpu.SemaphoreType.DMA,
          pltpu.SemaphoreType.DMA,
      ],
      compiler_params=pltpu.CompilerParams(use_tc_tiling_on_sc=False),
  )
  def msg_kernel(gtab, idxrc, sum_out, sq_out,
                 rbuf, sqbuf, ijb, zbuf, asum, asq, gsem, ssem):
    c = lax.axis_index("c")
    s = lax.axis_index("s")
    zeros16 = jnp.zeros((L,), F32)
    for r in range(32):
      for j in range(4):
        zbuf[r, pl.ds(j * L, L)] = zeros16

    r0 = s * rows_per_tile

    def zero_acc(k, carry):
      pltpu.sync_copy(zbuf, asum.at[pl.ds(r0 + k * 32, 32)])
      pltpu.sync_copy(zbuf, asq.at[pl.ds(r0 + k * 32, 32)])
      return carry
    lax.fori_loop(0, rows_per_tile // 32, zero_acc, 0)
    plsc.subcore_barrier()

    cbase = s * n_chunks

    def gather(rowk, br):
      pltpu.async_copy(gtab.at[ijb.at[rowk, c]], rbuf.at[br], gsem)

    def wait_gather():
      pltpu.make_async_copy(gtab.at[ijb.at[0, 0]], rbuf.at[0], gsem).wait()

    def square(br, b2):
      def sq(rr, carry):
        for u in range(4):
          for v in range(4):
            x = rbuf[br, rr * 4 + u, pl.ds(v * L, L)]
            sqbuf[b2, rr * 4 + u, pl.ds(v * L, L)] = x * x
        return carry
      lax.fori_loop(0, CHUNK // 4, sq, 0)

    def scat(rowk, br, b2):
      pltpu.async_copy(rbuf.at[br], asum.at[ijb.at[rowk, 2]], ssem, add=True)
      pltpu.async_copy(sqbuf.at[b2], asq.at[ijb.at[rowk, 2]], ssem, add=True)

    def drain():
      pltpu.make_async_copy(rbuf.at[0], asum.at[ijb.at[0, 2]], ssem).wait()
      pltpu.make_async_copy(sqbuf.at[0], asq.at[ijb.at[0, 2]], ssem).wait()

    pltpu.sync_copy(idxrc.at[pl.ds(cbase, BLK)], ijb.at[pl.ds(0, BLK)])
    gather(0, 0)
    gather(1, 1)

    def body(j, carry):
      h = (j % 2) * BLK
      h2 = ((j + 1) % 2) * BLK
      last = j >= n_blocks - 1
      for p in range(BLK):
        br = p % 3
        b2 = p % 2
        if p == 0:
          @pl.when(j > 0)
          def _():
            drain()
        else:
          drain()
        if p < BLK - 2:
          gather(h + p + 2, (p + 2) % 3)
        else:
          @pl.when(jnp.logical_not(last))
          def _():
            gather(h2 + p - (BLK - 2), (p + 2) % 3)
        if p == 0:
          @pl.when(jnp.logical_not(last))
          def _():
            pltpu.sync_copy(idxrc.at[pl.ds(cbase + (j + 1) * BLK, BLK)],
                            ijb.at[pl.ds(h2, BLK)])
        wait_gather()
        square(br, b2)
        scat(h + p, br, b2)
      return carry
    lax.fori_loop(0, n_blocks, body, 0)
    drain()
    plsc.subcore_barrier()

    def out_body(k, carry):
      pltpu.sync_copy(asum.at[pl.ds(r0 + k * 32, 32)], zbuf)
      pltpu.sync_copy(zbuf, sum_out.at[c, pl.ds(r0 + k * 32, 32)])
      pltpu.sync_copy(asq.at[pl.ds(r0 + k * 32, 32)], zbuf)
      pltpu.sync_copy(zbuf, sq_out.at[c, pl.ds(r0 + k * 32, 32)])
      return carry
    lax.fori_loop(0, rows_per_tile // 32, out_body, 0)

  return msg_kernel


def _dense_body(h_ref, s_ref, q_ref, dd_ref, w_ref, b_ref, g_ref, be_ref,
                o_ref):
  dd = dd_ref[...]
  sm = s_ref[...] * dd
  fm = 0.5 * (sm * sm - (dd * dd) * q_ref[...])
  x = jnp.concatenate([h_ref[...], sm, fm], axis=1)
  y = lax.dot_general(x, w_ref[...], (((1,), (0,)), ((), ())),
                      precision=lax.Precision.HIGHEST,
                      preferred_element_type=F32)
  y = y + b_ref[...]
  mu = jnp.mean(y, axis=1, keepdims=True)
  var = jnp.mean((y - mu) * (y - mu), axis=1, keepdims=True)
  ln = (y - mu) * lax.rsqrt(var + 1e-5) * g_ref[...] + be_ref[...]
  o_ref[...] = jnp.maximum(ln, 0.0)


def _make_dense_kernel(n, d, bn):
  return pl.pallas_call(
      _dense_body,
      grid=(n // bn,),
      in_specs=[
          pl.BlockSpec((bn, d), lambda i: (i, 0)),
          pl.BlockSpec((bn, d), lambda i: (i, 0)),
          pl.BlockSpec((bn, d), lambda i: (i, 0)),
          pl.BlockSpec((bn, 1), lambda i: (i, 0)),
          pl.BlockSpec((3 * d, d), lambda i: (0, 0)),
          pl.BlockSpec((1, d), lambda i: (0, 0)),
          pl.BlockSpec((1, d), lambda i: (0, 0)),
          pl.BlockSpec((1, d), lambda i: (0, 0)),
      ],
      out_specs=pl.BlockSpec((bn, d), lambda i: (i, 0)),
      out_shape=jax.ShapeDtypeStruct((n, d), F32),
  )


def kernel(h, edge_index, W_self, b_self, W_lin, b_lin, W_fm, b_fm,
           gamma, beta):
  n, d = h.shape
  e = edge_index.shape[1]
  assert d == 128
  n_pad = ((n + 1023) // 1024) * 1024
  quant = NS * CHUNK * BLK
  e_pad = ((e + quant - 1) // quant) * quant
  pad = e_pad - e

  row = edge_index[0]
  col = edge_index[1]
  row_h = jnp.concatenate([row, jnp.full((pad,), n, I32)])
  col_p = jnp.concatenate([col, jnp.full((pad,), n, I32)])
  degidx = jnp.stack([row_h.reshape(-1, CHUNK), col_p.reshape(-1, CHUNK)],
                     axis=1)

  deg = _make_deg_kernel(e_pad, n_pad)(degidx).reshape(NC, n_pad)
  deg_src = deg[0, :n]
  deg_dst = deg[1, :n]
  dsi = jnp.where(deg_src > 0, lax.rsqrt(jnp.maximum(deg_src, 1.0)), 0.0)
  ddi = jnp.where(deg_dst > 0, lax.rsqrt(jnp.maximum(deg_dst, 1.0)), 0.0)

  g = h * dsi[:, None]
  zrows = jnp.zeros((n_pad - n, 64), F32)
  gtab = jnp.concatenate([g[:, :64], zrows, g[:, 64:], zrows], axis=0)

  row_g = jnp.concatenate([row, jnp.zeros((pad,), I32)]).reshape(-1, CHUNK)
  idxrc = jnp.stack([row_g, row_g + n_pad, col_p.reshape(-1, CHUNK)], axis=1)

  sums, sqs = _make_msg_kernel(e_pad, n_pad)(gtab, idxrc)
  s_full = sums[:, :n, :].transpose(1, 0, 2).reshape(n, d)
  q_full = sqs[:, :n, :].transpose(1, 0, 2).reshape(n, d)

  w_cat = jnp.concatenate([W_self.T, W_lin.T, W_fm.T], axis=0)
  b_sum = (b_self + b_lin + b_fm)[None, :]

  return _make_dense_kernel(n, d, 1000)(
      h, s_full, q_full, ddi[:, None], w_cat, b_sum,
      gamma[None, :], beta[None, :])


# speedup vs baseline: 1.1312x; 1.1312x over previous
import functools

import jax
import jax.numpy as jnp
from jax import lax
from jax.experimental import pallas as pl
from jax.experimental.pallas import tpu as pltpu
from jax.experimental.pallas import tpu_sc as plsc

F32 = jnp.float32
I32 = jnp.int32

NC = 2
NS = 16
L = 16
CHUNK = 128
BLK = 4


def _sc_mesh():
  return plsc.VectorSubcoreMesh(core_axis_name="c", subcore_axis_name="s")


def _make_deg_kernel(e_pad, n_pad):
  per_tile = e_pad // NS
  n_chunks = per_tile // CHUNK
  n_blocks = n_chunks // BLK
  rows_per_tile = n_pad // NS

  @functools.partial(
      pl.kernel,
      out_type=jax.ShapeDtypeStruct((NC * n_pad,), F32),
      mesh=_sc_mesh(),
      scratch_types=[
          pltpu.VMEM((CHUNK,), F32),
          pltpu.VMEM((2 * BLK, 3, CHUNK), I32),
          pltpu.VMEM((rows_per_tile,), F32),
          pltpu.VMEM_SHARED((n_pad,), F32),
          pltpu.SemaphoreType.DMA,
      ],
      compiler_params=pltpu.CompilerParams(use_tc_tiling_on_sc=False),
  )
  def deg_kernel(degidx, deg_out, onesbuf, ijb, tbuf, acc, ssem):
    c = lax.axis_index("c")
    s = lax.axis_index("s")
    zeros16 = jnp.zeros((L,), F32)
    ones16 = jnp.ones((L,), F32)
    for j in range(CHUNK // L):
      onesbuf[pl.ds(j * L, L)] = ones16

    def zt(r, carry):
      tbuf[pl.ds(r * L, L)] = zeros16
      return carry
    lax.fori_loop(0, rows_per_tile // L, zt, 0)

    r0 = s * rows_per_tile
    pltpu.sync_copy(tbuf, acc.at[pl.ds(r0, rows_per_tile)])
    plsc.subcore_barrier()

    cbase = s * n_chunks

    sel = c * 2

    def scat(rowk):
      pltpu.async_copy(onesbuf, acc.at[ijb.at[rowk, sel]], ssem, add=True)

    def drain():
      pltpu.make_async_copy(onesbuf, acc.at[ijb.at[0, 0]], ssem).wait()

    pltpu.sync_copy(degidx.at[pl.ds(cbase, BLK)], ijb.at[pl.ds(0, BLK)])

    def body(j, carry):
      h = (j % 2) * BLK
      h2 = ((j + 1) % 2) * BLK
      for p in range(BLK):
        if p < 2:
          @pl.when(j > 0)
          def _():
            drain()
        else:
          drain()
        scat(h + p)
        if p == 1:
          @pl.when(j < n_blocks - 1)
          def _():
            pltpu.sync_copy(degidx.at[pl.ds(cbase + (j + 1) * BLK, BLK)],
                            ijb.at[pl.ds(h2, BLK)])
      return carry
    lax.fori_loop(0, n_blocks, body, 0)
    drain()
    drain()
    plsc.subcore_barrier()

    pltpu.sync_copy(acc.at[pl.ds(r0, rows_per_tile)], tbuf)
    pltpu.sync_copy(tbuf, deg_out.at[pl.ds(c * n_pad + r0, rows_per_tile)])

  return deg_kernel


def _make_msg_kernel(e_pad, n_pad):
  per_tile = e_pad // NS
  n_chunks = per_tile // CHUNK
  n_blocks = n_chunks // BLK
  rows_per_tile = n_pad // NS

  @functools.partial(
      pl.kernel,
      out_type=(jax.ShapeDtypeStruct((NC, n_pad, 64), F32),
                jax.ShapeDtypeStruct((NC, n_pad, 64), F32)),
      mesh=_sc_mesh(),
      scratch_types=[
          pltpu.VMEM((2, CHUNK, 64), F32),
          pltpu.VMEM((2, CHUNK, 64), F32),
          pltpu.VMEM((2 * BLK, 3, CHUNK), I32),
          pltpu.VMEM((32, 64), F32),
          pltpu.VMEM_SHARED((n_pad, 64), F32),
          pltpu.VMEM_SHARED((n_pad, 64), F32),
          pltpu.SemaphoreType.DMA,
          pltpu.SemaphoreType.DMA,
      ],
      compiler_params=pltpu.CompilerParams(use_tc_tiling_on_sc=False),
  )
  def msg_kernel(gtab, idxrc, sum_out, sq_out,
                 rbuf, sqbuf, ijb, zbuf, asum, asq, gsem, ssem):
    c = lax.axis_index("c")
    s = lax.axis_index("s")
    zeros16 = jnp.zeros((L,), F32)
    for r in range(32):
      for j in range(4):
        zbuf[r, pl.ds(j * L, L)] = zeros16

    r0 = s * rows_per_tile

    def zero_acc(k, carry):
      pltpu.sync_copy(zbuf, asum.at[pl.ds(r0 + k * 32, 32)])
      pltpu.sync_copy(zbuf, asq.at[pl.ds(r0 + k * 32, 32)])
      return carry
    lax.fori_loop(0, rows_per_tile // 32, zero_acc, 0)
    plsc.subcore_barrier()

    cbase = s * n_chunks

    def gather(rowk, br):
      pltpu.async_copy(gtab.at[ijb.at[rowk, c]], rbuf.at[br], gsem)

    def wait_gather():
      pltpu.make_async_copy(gtab.at[ijb.at[0, 0]], rbuf.at[0], gsem).wait()

    def square(b):
      def sq(rr, carry):
        for u in range(4):
          for v in range(4):
            x = rbuf[b, rr * 4 + u, pl.ds(v * L, L)]
            sqbuf[b, rr * 4 + u, pl.ds(v * L, L)] = x * x
        return carry
      lax.fori_loop(0, CHUNK // 4, sq, 0)

    def scat(rowk, b):
      pltpu.async_copy(rbuf.at[b], asum.at[ijb.at[rowk, 2]], ssem, add=True)
      pltpu.async_copy(sqbuf.at[b], asq.at[ijb.at[rowk, 2]], ssem, add=True)

    def drain():
      pltpu.make_async_copy(rbuf.at[0], asum.at[ijb.at[0, 2]], ssem).wait()
      pltpu.make_async_copy(sqbuf.at[0], asq.at[ijb.at[0, 2]], ssem).wait()

    pltpu.sync_copy(idxrc.at[pl.ds(cbase, BLK)], ijb.at[pl.ds(0, BLK)])
    gather(0, 0)

    def body(j, carry):
      h = (j % 2) * BLK
      h2 = ((j + 1) % 2) * BLK
      last = j >= n_blocks - 1
      for p in range(BLK):
        b = p % 2
        nb = 1 - b
        if p == 0:
          @pl.when(j > 0)
          def _():
            drain()
        else:
          drain()
        if p < BLK - 1:
          gather(h + p + 1, nb)
        else:
          @pl.when(jnp.logical_not(last))
          def _():
            gather(h2, nb)
        wait_gather()
        square(b)
        scat(h + p, b)
        if p == 1:
          @pl.when(jnp.logical_not(last))
          def _():
            pltpu.sync_copy(idxrc.at[pl.ds(cbase + (j + 1) * BLK, BLK)],
                            ijb.at[pl.ds(h2, BLK)])
      return carry
    lax.fori_loop(0, n_blocks, body, 0)
    drain()
    plsc.subcore_barrier()

    def out_body(k, carry):
      pltpu.sync_copy(asum.at[pl.ds(r0 + k * 32, 32)], zbuf)
      pltpu.sync_copy(zbuf, sum_out.at[c, pl.ds(r0 + k * 32, 32)])
      pltpu.sync_copy(asq.at[pl.ds(r0 + k * 32, 32)], zbuf)
      pltpu.sync_copy(zbuf, sq_out.at[c, pl.ds(r0 + k * 32, 32)])
      return carry
    lax.fori_loop(0, rows_per_tile // 32, out_body, 0)

  return msg_kernel


def _dense_body(h_ref, s_ref, q_ref, dd_ref, w_ref, b_ref, g_ref, be_ref,
                o_ref):
  dd = dd_ref[...]
  s_cat = jnp.concatenate([s_ref[0], s_ref[1]], axis=1)
  q_cat = jnp.concatenate([q_ref[0], q_ref[1]], axis=1)
  sm = s_cat * dd
  fm = 0.5 * (sm * sm - (dd * dd) * q_cat)
  x = jnp.concatenate([h_ref[...], sm, fm], axis=1)
  y = lax.dot_general(x, w_ref[...], (((1,), (0,)), ((), ())),
                      precision=lax.Precision.HIGHEST,
                      preferred_element_type=F32)
  y = y + b_ref[...]
  mu = jnp.mean(y, axis=1, keepdims=True)
  var = jnp.mean((y - mu) * (y - mu), axis=1, keepdims=True)
  ln = (y - mu) * lax.rsqrt(var + 1e-5) * g_ref[...] + be_ref[...]
  o_ref[...] = jnp.maximum(ln, 0.0)


def _make_dense_kernel(n, d, bn):
  return pl.pallas_call(
      _dense_body,
      grid=(n // bn,),
      in_specs=[
          pl.BlockSpec((bn, d), lambda i: (i, 0)),
          pl.BlockSpec((NC, bn, 64), lambda i: (0, i, 0)),
          pl.BlockSpec((NC, bn, 64), lambda i: (0, i, 0)),
          pl.BlockSpec((bn, 1), lambda i: (i, 0)),
          pl.BlockSpec((3 * d, d), lambda i: (0, 0)),
          pl.BlockSpec((1, d), lambda i: (0, 0)),
          pl.BlockSpec((1, d), lambda i: (0, 0)),
          pl.BlockSpec((1, d), lambda i: (0, 0)),
      ],
      out_specs=pl.BlockSpec((bn, d), lambda i: (i, 0)),
      out_shape=jax.ShapeDtypeStruct((n, d), F32),
  )


def kernel(h, edge_index, W_self, b_self, W_lin, b_lin, W_fm, b_fm,
           gamma, beta):
  n, d = h.shape
  e = edge_index.shape[1]
  assert d == 128
  n_pad = ((n + 1023) // 1024) * 1024
  quant = NS * CHUNK * BLK
  e_pad = ((e + quant - 1) // quant) * quant
  pad = e_pad - e

  row = edge_index[0]
  col = edge_index[1]
  row_g = jnp.concatenate([row, jnp.zeros((pad,), I32)]).reshape(-1, CHUNK)
  col_p = jnp.concatenate([col, jnp.full((pad,), n, I32)]).reshape(-1, CHUNK)
  idxrc = jnp.stack([row_g, row_g + n_pad, col_p], axis=1)

  deg = _make_deg_kernel(e_pad, n_pad)(idxrc).reshape(NC, n_pad)
  deg_src = deg[0, :n].at[0].add(-float(pad))
  deg_dst = deg[1, :n]
  dsi = jnp.where(deg_src > 0, lax.rsqrt(jnp.maximum(deg_src, 1.0)), 0.0)
  ddi = jnp.where(deg_dst > 0, lax.rsqrt(jnp.maximum(deg_dst, 1.0)), 0.0)

  g = h * dsi[:, None]
  zrows = jnp.zeros((n_pad - n, 64), F32)
  gtab = jnp.concatenate([g[:, :64], zrows, g[:, 64:], zrows], axis=0)

  sums, sqs = _make_msg_kernel(e_pad, n_pad)(gtab, idxrc)

  w_cat = jnp.concatenate([W_self.T, W_lin.T, W_fm.T], axis=0)
  b_sum = (b_self + b_lin + b_fm)[None, :]

  return _make_dense_kernel(n, d, 1000)(
      h, sums, sqs, ddi[:, None], w_cat, b_sum,
      gamma[None, :], beta[None, :])
